# initial kernel scaffold (unmeasured)
import jax
import jax.numpy as jnp
from jax import lax
from jax.experimental import pallas as pl
from jax.experimental.pallas import tpu as pltpu

N_DEV = 4


def kernel(x, w_mat):
    m_total, k_shard = x.shape
    _, n = w_mat.shape
    m_chunk = m_total // N_DEV

    def body(x_ref, w_ref, out_ref, comm_ref, send_sems, recv_sems,
             amax_ref, amax_send_sems, amax_recv_sems):
        d = lax.axis_index("i")
        left = lax.rem(d + N_DEV - 1, N_DEV)
        right = lax.rem(d + 1, N_DEV)

        barrier_sem = pltpu.get_barrier_semaphore()
        for nbr in (left, right):
            pl.semaphore_signal(
                barrier_sem, inc=1,
                device_id=(nbr,), device_id_type=pl.DeviceIdType.MESH,
            )
        pl.semaphore_wait(barrier_sem, 2)

        def mm_chunk(c):
            xc = x_ref[pl.ds(c * m_chunk, m_chunk), :]
            return lax.dot_general(
                xc, w_ref[:, :], (((1,), (0,)), ((), ())),
                preferred_element_type=jnp.float32,
            )

        comm_ref[0] = mm_chunk(lax.rem(d + 3, N_DEV))

        for s in range(N_DEV - 1):
            rdma = pltpu.make_async_remote_copy(
                src_ref=comm_ref.at[s],
                dst_ref=comm_ref.at[s + 1],
                send_sem=send_sems.at[s],
                recv_sem=recv_sems.at[s + 1],
                device_id=(right,),
                device_id_type=pl.DeviceIdType.MESH,
            )
            rdma.start()
            own = mm_chunk(lax.rem(d + 2 - s + N_DEV, N_DEV))
            rdma.wait()
            if s < N_DEV - 2:
                comm_ref[s + 1] = comm_ref[s + 1] + own
            else:
                out_ref[:, :] = comm_ref[s + 1] + own

        local_amax = jnp.max(jnp.abs(out_ref[:, :]))
        amax_ref[d] = jnp.broadcast_to(local_amax, (8, 128)).astype(jnp.float32)

        send_descs = []
        for off in range(1, N_DEV):
            p = lax.rem(d + off, N_DEV)
            rd = pltpu.make_async_remote_copy(
                src_ref=amax_ref.at[d],
                dst_ref=amax_ref.at[d],
                send_sem=amax_send_sems.at[off - 1],
                recv_sem=amax_recv_sems.at[d],
                device_id=(p,),
                device_id_type=pl.DeviceIdType.MESH,
            )
            rd.start()
            send_descs.append(rd)
        for off in range(1, N_DEV):
            p = lax.rem(d + off, N_DEV)
            rwait = pltpu.make_async_remote_copy(
                src_ref=amax_ref.at[d],
                dst_ref=amax_ref.at[p],
                send_sem=amax_send_sems.at[off - 1],
                recv_sem=amax_recv_sems.at[p],
                device_id=(p,),
                device_id_type=pl.DeviceIdType.MESH,
            )
            rwait.wait_recv()
        for rd in send_descs:
            rd.wait_send()

        g_amax = jnp.max(amax_ref[:, :, :])

        scale = g_amax / 448.0
        q = (out_ref[:, :] / scale).astype(jnp.float8_e4m3fn)
        out_ref[:, :] = q.astype(jnp.float32) * scale

    return pl.pallas_call(
        body,
        out_shape=jax.ShapeDtypeStruct((m_chunk, n), jnp.float32),
        in_specs=[
            pl.BlockSpec(memory_space=pltpu.VMEM),
            pl.BlockSpec(memory_space=pltpu.VMEM),
        ],
        out_specs=pl.BlockSpec(memory_space=pltpu.VMEM),
        scratch_shapes=[
            pltpu.VMEM((N_DEV, m_chunk, n), jnp.float32),
            pltpu.SemaphoreType.DMA((N_DEV - 1,)),
            pltpu.SemaphoreType.DMA((N_DEV,)),
            pltpu.VMEM((N_DEV, 8, 128), jnp.float32),
            pltpu.SemaphoreType.DMA((N_DEV - 1,)),
            pltpu.SemaphoreType.DMA((N_DEV,)),
        ],
        compiler_params=pltpu.CompilerParams(
            collective_id=0,
            vmem_limit_bytes=128 * 1024 * 1024,
        ),
    )(x, w_mat)


# baseline (device time: 305171 ns/iter reference)
import jax
import jax.numpy as jnp
from jax import lax
from jax.experimental import pallas as pl
from jax.experimental.pallas import tpu as pltpu

N_DEV = 4


def kernel(x, w_mat):
    m_total, k_shard = x.shape
    _, n = w_mat.shape
    m_chunk = m_total // N_DEV

    def body(x_ref, w_ref, out_ref, stage_ref, load_sem,
             comm_ref, send_sems, recv_sems,
             amax_ref, amax_send_sems, amax_recv_sems):
        d = lax.axis_index("i")
        left = lax.rem(d + N_DEV - 1, N_DEV)
        right = lax.rem(d + 1, N_DEV)

        barrier_sem = pltpu.get_barrier_semaphore()
        for nbr in (left, right):
            pl.semaphore_signal(
                barrier_sem, inc=1,
                device_id=(nbr,), device_id_type=pl.DeviceIdType.MESH,
            )
        pl.semaphore_wait(barrier_sem, 2)

        def load_chunk(c):
            cp = pltpu.make_async_copy(
                x_ref.at[pl.ds(c * m_chunk, m_chunk), :],
                stage_ref,
                load_sem,
            )
            cp.start()
            cp.wait()

        def mm_stage():
            return lax.dot_general(
                stage_ref[:, :], w_ref[:, :], (((1,), (0,)), ((), ())),
                preferred_element_type=jnp.float32,
            )

        load_chunk(lax.rem(d + 3, N_DEV))
        comm_ref[0] = mm_stage()

        for s in range(N_DEV - 1):
            send_slot = s
            recv_slot = (s + 1) % 3
            rdma = pltpu.make_async_remote_copy(
                src_ref=comm_ref.at[send_slot],
                dst_ref=comm_ref.at[recv_slot],
                send_sem=send_sems.at[send_slot],
                recv_sem=recv_sems.at[recv_slot],
                device_id=(right,),
                device_id_type=pl.DeviceIdType.MESH,
            )
            rdma.start()
            load_chunk(lax.rem(d + 2 - s + N_DEV, N_DEV))
            own = mm_stage()
            rdma.wait()
            if s < N_DEV - 2:
                comm_ref[recv_slot] = comm_ref[recv_slot] + own
            else:
                out_ref[:, :] = comm_ref[recv_slot] + own

        local_amax = jnp.max(jnp.abs(out_ref[:, :]))
        amax_ref[d] = jnp.broadcast_to(local_amax, (8, 128)).astype(jnp.float32)

        send_descs = []
        for off in range(1, N_DEV):
            p = lax.rem(d + off, N_DEV)
            rd = pltpu.make_async_remote_copy(
                src_ref=amax_ref.at[d],
                dst_ref=amax_ref.at[d],
                send_sem=amax_send_sems.at[off - 1],
                recv_sem=amax_recv_sems.at[d],
                device_id=(p,),
                device_id_type=pl.DeviceIdType.MESH,
            )
            rd.start()
            send_descs.append(rd)
        for off in range(1, N_DEV):
            p = lax.rem(d + off, N_DEV)
            rwait = pltpu.make_async_remote_copy(
                src_ref=amax_ref.at[d],
                dst_ref=amax_ref.at[p],
                send_sem=amax_send_sems.at[off - 1],
                recv_sem=amax_recv_sems.at[p],
                device_id=(p,),
                device_id_type=pl.DeviceIdType.MESH,
            )
            rwait.wait_recv()
        for rd in send_descs:
            rd.wait_send()

        g_amax = jnp.max(amax_ref[:, :, :])

        scale = g_amax / 448.0
        q = (out_ref[:, :] / scale).astype(jnp.float8_e4m3fn)
        out_ref[:, :] = q.astype(jnp.float32) * scale

    return pl.pallas_call(
        body,
        out_shape=jax.ShapeDtypeStruct((m_chunk, n), jnp.float32),
        in_specs=[
            pl.BlockSpec(memory_space=pl.ANY),
            pl.BlockSpec(memory_space=pltpu.VMEM),
        ],
        out_specs=pl.BlockSpec(memory_space=pltpu.VMEM),
        scratch_shapes=[
            pltpu.VMEM((m_chunk, k_shard), jnp.float32),
            pltpu.SemaphoreType.DMA,
            pltpu.VMEM((3, m_chunk, n), jnp.float32),
            pltpu.SemaphoreType.DMA((3,)),
            pltpu.SemaphoreType.DMA((3,)),
            pltpu.VMEM((N_DEV, 8, 128), jnp.float32),
            pltpu.SemaphoreType.DMA((N_DEV - 1,)),
            pltpu.SemaphoreType.DMA((N_DEV,)),
        ],
        compiler_params=pltpu.CompilerParams(
            collective_id=0,
            vmem_limit_bytes=128 * 1024 * 1024,
        ),
    )(x, w_mat)


# device time: 172786 ns/iter; 1.7662x vs baseline; 1.7662x over previous
import jax
import jax.numpy as jnp
from jax import lax
from jax.experimental import pallas as pl
from jax.experimental.pallas import tpu as pltpu

N_DEV = 4


def kernel(x, w_mat):
    m_total, k_shard = x.shape
    _, n = w_mat.shape
    m_chunk = m_total // N_DEV
    n_half = n // 2

    def body(x_ref, w_ref, out_ref, stage_ref, load_sems,
             commR_ref, sendR_sems, recvR_sems,
             commL_ref, sendL_sems, recvL_sems,
             amax_ref, amax_send_sems, amax_recv_sems):
        d = lax.axis_index("i")
        left = lax.rem(d + N_DEV - 1, N_DEV)
        right = lax.rem(d + 1, N_DEV)

        barrier_sem = pltpu.get_barrier_semaphore()
        for nbr in (left, right):
            pl.semaphore_signal(
                barrier_sem, inc=1,
                device_id=(nbr,), device_id_type=pl.DeviceIdType.MESH,
            )
        pl.semaphore_wait(barrier_sem, 2)

        def load_chunk(c, slot):
            cp = pltpu.make_async_copy(
                x_ref.at[pl.ds(c * m_chunk, m_chunk), :],
                stage_ref.at[slot],
                load_sems.at[slot],
            )
            cp.start()
            cp.wait()

        def mm(slot, half):
            return lax.dot_general(
                stage_ref[slot],
                w_ref[:, pl.ds(half * n_half, n_half)],
                (((1,), (0,)), ((), ())),
                preferred_element_type=jnp.float32,
            )

        load_chunk(lax.rem(d + 3, N_DEV), 0)
        commR_ref[0] = mm(0, 0)
        load_chunk(lax.rem(d + 1, N_DEV), 1)
        commL_ref[0] = mm(1, 1)

        for s in range(N_DEV - 1):
            send_slot = s
            recv_slot = (s + 1) % 3
            rdmaR = pltpu.make_async_remote_copy(
                src_ref=commR_ref.at[send_slot],
                dst_ref=commR_ref.at[recv_slot],
                send_sem=sendR_sems.at[send_slot],
                recv_sem=recvR_sems.at[recv_slot],
                device_id=(right,),
                device_id_type=pl.DeviceIdType.MESH,
            )
            rdmaL = pltpu.make_async_remote_copy(
                src_ref=commL_ref.at[send_slot],
                dst_ref=commL_ref.at[recv_slot],
                send_sem=sendL_sems.at[send_slot],
                recv_sem=recvL_sems.at[recv_slot],
                device_id=(left,),
                device_id_type=pl.DeviceIdType.MESH,
            )
            rdmaR.start()
            rdmaL.start()

            cR = lax.rem(d + 2 - s + N_DEV, N_DEV)
            cL = lax.rem(d + 2 + s, N_DEV)
            if s == 1:
                load_chunk(cR, 0)
                ownR = mm(0, 0)
                load_chunk(cL, 1)
                ownL = mm(1, 1)
            else:
                load_chunk(cR, 0)
                ownR = mm(0, 0)
                ownL = mm(0, 1)

            rdmaR.wait()
            rdmaL.wait()
            if s < N_DEV - 2:
                commR_ref[recv_slot] = commR_ref[recv_slot] + ownR
                commL_ref[recv_slot] = commL_ref[recv_slot] + ownL
            else:
                out_ref[:, :n_half] = commR_ref[recv_slot] + ownR
                out_ref[:, n_half:] = commL_ref[recv_slot] + ownL

        local_amax = jnp.max(jnp.abs(out_ref[:, :]))
        amax_ref[d] = jnp.broadcast_to(local_amax, (8, 128)).astype(jnp.float32)

        send_descs = []
        for off in range(1, N_DEV):
            p = lax.rem(d + off, N_DEV)
            rd = pltpu.make_async_remote_copy(
                src_ref=amax_ref.at[d],
                dst_ref=amax_ref.at[d],
                send_sem=amax_send_sems.at[off - 1],
                recv_sem=amax_recv_sems.at[d],
                device_id=(p,),
                device_id_type=pl.DeviceIdType.MESH,
            )
            rd.start()
            send_descs.append(rd)
        for off in range(1, N_DEV):
            p = lax.rem(d + off, N_DEV)
            rwait = pltpu.make_async_remote_copy(
                src_ref=amax_ref.at[d],
                dst_ref=amax_ref.at[p],
                send_sem=amax_send_sems.at[off - 1],
                recv_sem=amax_recv_sems.at[p],
                device_id=(p,),
                device_id_type=pl.DeviceIdType.MESH,
            )
            rwait.wait_recv()
        for rd in send_descs:
            rd.wait_send()

        g_amax = jnp.max(amax_ref[:, :, :])

        scale = g_amax / 448.0
        q = (out_ref[:, :] / scale).astype(jnp.float8_e4m3fn)
        out_ref[:, :] = q.astype(jnp.float32) * scale

    return pl.pallas_call(
        body,
        out_shape=jax.ShapeDtypeStruct((m_chunk, n), jnp.float32),
        in_specs=[
            pl.BlockSpec(memory_space=pl.ANY),
            pl.BlockSpec(memory_space=pltpu.VMEM),
        ],
        out_specs=pl.BlockSpec(memory_space=pltpu.VMEM),
        scratch_shapes=[
            pltpu.VMEM((2, m_chunk, k_shard), jnp.float32),
            pltpu.SemaphoreType.DMA((2,)),
            pltpu.VMEM((3, m_chunk, n_half), jnp.float32),
            pltpu.SemaphoreType.DMA((3,)),
            pltpu.SemaphoreType.DMA((3,)),
            pltpu.VMEM((3, m_chunk, n_half), jnp.float32),
            pltpu.SemaphoreType.DMA((3,)),
            pltpu.SemaphoreType.DMA((3,)),
            pltpu.VMEM((N_DEV, 8, 128), jnp.float32),
            pltpu.SemaphoreType.DMA((N_DEV - 1,)),
            pltpu.SemaphoreType.DMA((N_DEV,)),
        ],
        compiler_params=pltpu.CompilerParams(
            collective_id=0,
            vmem_limit_bytes=128 * 1024 * 1024,
        ),
    )(x, w_mat)


# device time: 165042 ns/iter; 1.8491x vs baseline; 1.0469x over previous
import jax
import jax.numpy as jnp
from jax import lax
from jax.experimental import pallas as pl
from jax.experimental.pallas import tpu as pltpu

N_DEV = 4
N_SUB = 2


def kernel(x, w_mat):
    m_total, k_shard = x.shape
    _, n = w_mat.shape
    m_chunk = m_total // N_DEV
    n_half = n // 2
    n_sub = n_half // N_SUB

    def body(x_ref, w_ref, out_ref, stage_ref, load_sems,
             commR_ref, sendR_sems, recvR_sems,
             commL_ref, sendL_sems, recvL_sems,
             amax_ref, amax_send_sems, amax_recv_sems):
        d = lax.axis_index("i")
        left = lax.rem(d + N_DEV - 1, N_DEV)
        right = lax.rem(d + 1, N_DEV)

        barrier_sem = pltpu.get_barrier_semaphore()
        for nbr in (left, right):
            pl.semaphore_signal(
                barrier_sem, inc=1,
                device_id=(nbr,), device_id_type=pl.DeviceIdType.MESH,
            )
        pl.semaphore_wait(barrier_sem, 2)

        def load_chunk(c, slot):
            cp = pltpu.make_async_copy(
                x_ref.at[pl.ds(c * m_chunk, m_chunk), :],
                stage_ref.at[slot],
                load_sems.at[slot],
            )
            cp.start()
            cp.wait()

        def mm_sub(slot, dirn, h):
            col0 = dirn * n_half + h * n_sub
            return lax.dot_general(
                stage_ref[slot], w_ref[:, col0:col0 + n_sub],
                (((1,), (0,)), ((), ())),
                preferred_element_type=jnp.float32,
            )

        def ring_send(comm_ref, send_sems, recv_sems, tgt, dirn, s, h):
            if s < N_DEV - 2:
                dst = comm_ref.at[s + 1, :, h * n_sub:(h + 1) * n_sub]
            else:
                col0 = dirn * n_half + h * n_sub
                dst = out_ref.at[:, col0:col0 + n_sub]
            return pltpu.make_async_remote_copy(
                src_ref=comm_ref.at[s, :, h * n_sub:(h + 1) * n_sub],
                dst_ref=dst,
                send_sem=send_sems.at[s, h],
                recv_sem=recv_sems.at[s, h],
                device_id=(tgt,),
                device_id_type=pl.DeviceIdType.MESH,
            )

        def ring_recv_wait(comm_ref, send_sems, recv_sems, tgt, dirn, s, h):
            ring_send(comm_ref, send_sems, recv_sems, tgt, dirn, s, h).wait_recv()

        argsR = (commR_ref, sendR_sems, recvR_sems, right, 0)
        argsL = (commL_ref, sendL_sems, recvL_sems, left, 1)

        send_descs = []

        load_chunk(lax.rem(d + 3, N_DEV), 0)
        commR_ref[0, :, :n_sub] = mm_sub(0, 0, 0)
        rd = ring_send(*argsR, 0, 0); rd.start(); send_descs.append(rd)
        load_chunk(lax.rem(d + 1, N_DEV), 1)
        commL_ref[0, :, :n_sub] = mm_sub(1, 1, 0)
        rd = ring_send(*argsL, 0, 0); rd.start(); send_descs.append(rd)
        commR_ref[0, :, n_sub:] = mm_sub(0, 0, 1)
        rd = ring_send(*argsR, 0, 1); rd.start(); send_descs.append(rd)
        commL_ref[0, :, n_sub:] = mm_sub(1, 1, 1)
        rd = ring_send(*argsL, 0, 1); rd.start(); send_descs.append(rd)

        for s in range(N_DEV - 1):
            cR = lax.rem(d + 2 - s + N_DEV, N_DEV)
            cL = lax.rem(d + 2 + s, N_DEV)
            if s == 1:
                load_chunk(cR, 0)
                load_chunk(cL, 1)
                slotR, slotL = 0, 1
            else:
                load_chunk(cR, 0)
                slotR, slotL = 0, 0

            for h in range(N_SUB):
                ownR = mm_sub(slotR, 0, h)
                ownL = mm_sub(slotL, 1, h)
                lo, hi = h * n_sub, (h + 1) * n_sub
                ring_recv_wait(*argsR, s, h)
                if s < N_DEV - 2:
                    commR_ref[s + 1, :, lo:hi] = commR_ref[s + 1, :, lo:hi] + ownR
                    rd = ring_send(*argsR, s + 1, h)
                    rd.start(); send_descs.append(rd)
                else:
                    out_ref[:, lo:hi] = out_ref[:, lo:hi] + ownR
                ring_recv_wait(*argsL, s, h)
                llo, lhi = n_half + lo, n_half + hi
                if s < N_DEV - 2:
                    commL_ref[s + 1, :, lo:hi] = commL_ref[s + 1, :, lo:hi] + ownL
                    rd = ring_send(*argsL, s + 1, h)
                    rd.start(); send_descs.append(rd)
                else:
                    out_ref[:, llo:lhi] = out_ref[:, llo:lhi] + ownL

        for rd in send_descs:
            rd.wait_send()

        local_amax = jnp.max(jnp.abs(out_ref[:, :]))
        amax_ref[d] = jnp.broadcast_to(local_amax, (8, 128)).astype(jnp.float32)

        amax_descs = []
        for off in range(1, N_DEV):
            p = lax.rem(d + off, N_DEV)
            rd = pltpu.make_async_remote_copy(
                src_ref=amax_ref.at[d],
                dst_ref=amax_ref.at[d],
                send_sem=amax_send_sems.at[off - 1],
                recv_sem=amax_recv_sems.at[d],
                device_id=(p,),
                device_id_type=pl.DeviceIdType.MESH,
            )
            rd.start()
            amax_descs.append(rd)
        for off in range(1, N_DEV):
            p = lax.rem(d + off, N_DEV)
            rwait = pltpu.make_async_remote_copy(
                src_ref=amax_ref.at[d],
                dst_ref=amax_ref.at[p],
                send_sem=amax_send_sems.at[off - 1],
                recv_sem=amax_recv_sems.at[p],
                device_id=(p,),
                device_id_type=pl.DeviceIdType.MESH,
            )
            rwait.wait_recv()
        for rd in amax_descs:
            rd.wait_send()

        g_amax = jnp.max(amax_ref[:, :, :])

        scale = g_amax / 448.0
        q = (out_ref[:, :] / scale).astype(jnp.float8_e4m3fn)
        out_ref[:, :] = q.astype(jnp.float32) * scale

    return pl.pallas_call(
        body,
        out_shape=jax.ShapeDtypeStruct((m_chunk, n), jnp.float32),
        in_specs=[
            pl.BlockSpec(memory_space=pl.ANY),
            pl.BlockSpec(memory_space=pltpu.VMEM),
        ],
        out_specs=pl.BlockSpec(memory_space=pltpu.VMEM),
        scratch_shapes=[
            pltpu.VMEM((2, m_chunk, k_shard), jnp.float32),
            pltpu.SemaphoreType.DMA((2,)),
            pltpu.VMEM((3, m_chunk, n_half), jnp.float32),
            pltpu.SemaphoreType.DMA((3, N_SUB)),
            pltpu.SemaphoreType.DMA((3, N_SUB)),
            pltpu.VMEM((3, m_chunk, n_half), jnp.float32),
            pltpu.SemaphoreType.DMA((3, N_SUB)),
            pltpu.SemaphoreType.DMA((3, N_SUB)),
            pltpu.VMEM((N_DEV, 8, 128), jnp.float32),
            pltpu.SemaphoreType.DMA((N_DEV - 1,)),
            pltpu.SemaphoreType.DMA((N_DEV,)),
        ],
        compiler_params=pltpu.CompilerParams(
            collective_id=0,
            vmem_limit_bytes=128 * 1024 * 1024,
        ),
    )(x, w_mat)


# device time: 162593 ns/iter; 1.8769x vs baseline; 1.0151x over previous
import jax
import jax.numpy as jnp
from jax import lax
from jax.experimental import pallas as pl
from jax.experimental.pallas import tpu as pltpu

N_DEV = 4
N_SUB = 2


def kernel(x, w_mat):
    m_total, k_shard = x.shape
    _, n = w_mat.shape
    m_chunk = m_total // N_DEV
    n_half = n // 2
    n_sub = n_half // N_SUB

    def body(x_ref, w_ref, out_ref, stage_ref, load_sems,
             commR_ref, sendR_sems, recvR_sems,
             commL_ref, sendL_sems, recvL_sems,
             amax_ref, amax_send_sems, amax_recv_sems):
        d = lax.axis_index("i")
        left = lax.rem(d + N_DEV - 1, N_DEV)
        right = lax.rem(d + 1, N_DEV)

        barrier_sem = pltpu.get_barrier_semaphore()
        for nbr in (left, right):
            pl.semaphore_signal(
                barrier_sem, inc=1,
                device_id=(nbr,), device_id_type=pl.DeviceIdType.MESH,
            )
        pl.semaphore_wait(barrier_sem, 2)

        def load_chunk(c, slot):
            return pltpu.make_async_copy(
                x_ref.at[pl.ds(c * m_chunk, m_chunk), :],
                stage_ref.at[slot],
                load_sems.at[slot],
            )

        def mm_sub(slot, dirn, h):
            col0 = dirn * n_half + h * n_sub
            return lax.dot_general(
                stage_ref[slot], w_ref[:, col0:col0 + n_sub],
                (((1,), (0,)), ((), ())),
                preferred_element_type=jnp.float32,
            )

        def ring_send(comm_ref, send_sems, recv_sems, tgt, dirn, s, h):
            if s < N_DEV - 2:
                dst = comm_ref.at[s + 1, :, h * n_sub:(h + 1) * n_sub]
            else:
                col0 = dirn * n_half + h * n_sub
                dst = out_ref.at[:, col0:col0 + n_sub]
            return pltpu.make_async_remote_copy(
                src_ref=comm_ref.at[s, :, h * n_sub:(h + 1) * n_sub],
                dst_ref=dst,
                send_sem=send_sems.at[s, h],
                recv_sem=recv_sems.at[s, h],
                device_id=(tgt,),
                device_id_type=pl.DeviceIdType.MESH,
            )

        def ring_recv_wait(comm_ref, send_sems, recv_sems, tgt, dirn, s, h):
            ring_send(comm_ref, send_sems, recv_sems, tgt, dirn, s, h).wait_recv()

        argsR = (commR_ref, sendR_sems, recvR_sems, right, 0)
        argsL = (commL_ref, sendL_sems, recvL_sems, left, 1)

        send_descs = []

        def start_send(args, s, h):
            rd = ring_send(*args, s, h)
            rd.start()
            send_descs.append(rd)

        ld0 = load_chunk(lax.rem(d + 3, N_DEV), 0)
        ld1 = load_chunk(lax.rem(d + 1, N_DEV), 1)
        ld2 = load_chunk(lax.rem(d + 2, N_DEV), 2)
        ld0.start()
        ld1.start()
        ld2.start()

        ld0.wait()
        commR_ref[0, :, :n_sub] = mm_sub(0, 0, 0)
        start_send(argsR, 0, 0)
        ld1.wait()
        commL_ref[0, :, :n_sub] = mm_sub(1, 1, 0)
        start_send(argsL, 0, 0)
        commR_ref[0, :, n_sub:] = mm_sub(0, 0, 1)
        start_send(argsR, 0, 1)
        commL_ref[0, :, n_sub:] = mm_sub(1, 1, 1)
        start_send(argsL, 0, 1)

        local_amax = jnp.float32(0.0)
        for s in range(N_DEV - 1):
            if s == 0:
                ld2.wait()
                slotR = slotL = 2
            elif s == 1:
                slotR, slotL = 1, 0
            else:
                ld2b.wait()
                slotR = slotL = 2

            for h in range(N_SUB):
                ownR = mm_sub(slotR, 0, h)
                ownL = mm_sub(slotL, 1, h)
                if s == 0 and h == N_SUB - 1:
                    ld2b = load_chunk(d, 2)
                    ld2b.start()
                lo, hi = h * n_sub, (h + 1) * n_sub
                ring_recv_wait(*argsR, s, h)
                if s < N_DEV - 2:
                    commR_ref[s + 1, :, lo:hi] = commR_ref[s + 1, :, lo:hi] + ownR
                    start_send(argsR, s + 1, h)
                else:
                    finalR = out_ref[:, lo:hi] + ownR
                    out_ref[:, lo:hi] = finalR
                    local_amax = jnp.maximum(local_amax, jnp.max(jnp.abs(finalR)))
                ring_recv_wait(*argsL, s, h)
                llo, lhi = n_half + lo, n_half + hi
                if s < N_DEV - 2:
                    commL_ref[s + 1, :, lo:hi] = commL_ref[s + 1, :, lo:hi] + ownL
                    start_send(argsL, s + 1, h)
                else:
                    finalL = out_ref[:, llo:lhi] + ownL
                    out_ref[:, llo:lhi] = finalL
                    local_amax = jnp.maximum(local_amax, jnp.max(jnp.abs(finalL)))

        amax_ref[d] = jnp.broadcast_to(local_amax, (8, 128)).astype(jnp.float32)

        amax_descs = []
        for off in range(1, N_DEV):
            p = lax.rem(d + off, N_DEV)
            rd = pltpu.make_async_remote_copy(
                src_ref=amax_ref.at[d],
                dst_ref=amax_ref.at[d],
                send_sem=amax_send_sems.at[off - 1],
                recv_sem=amax_recv_sems.at[d],
                device_id=(p,),
                device_id_type=pl.DeviceIdType.MESH,
            )
            rd.start()
            amax_descs.append(rd)

        for rd in send_descs:
            rd.wait_send()

        for off in range(1, N_DEV):
            p = lax.rem(d + off, N_DEV)
            rwait = pltpu.make_async_remote_copy(
                src_ref=amax_ref.at[d],
                dst_ref=amax_ref.at[p],
                send_sem=amax_send_sems.at[off - 1],
                recv_sem=amax_recv_sems.at[p],
                device_id=(p,),
                device_id_type=pl.DeviceIdType.MESH,
            )
            rwait.wait_recv()
        for rd in amax_descs:
            rd.wait_send()

        g_amax = jnp.max(amax_ref[:, :, :])

        scale = g_amax / 448.0
        q = (out_ref[:, :] / scale).astype(jnp.float8_e4m3fn)
        out_ref[:, :] = q.astype(jnp.float32) * scale

    return pl.pallas_call(
        body,
        out_shape=jax.ShapeDtypeStruct((m_chunk, n), jnp.float32),
        in_specs=[
            pl.BlockSpec(memory_space=pl.ANY),
            pl.BlockSpec(memory_space=pltpu.VMEM),
        ],
        out_specs=pl.BlockSpec(memory_space=pltpu.VMEM),
        scratch_shapes=[
            pltpu.VMEM((3, m_chunk, k_shard), jnp.float32),
            pltpu.SemaphoreType.DMA((3,)),
            pltpu.VMEM((3, m_chunk, n_half), jnp.float32),
            pltpu.SemaphoreType.DMA((3, N_SUB)),
            pltpu.SemaphoreType.DMA((3, N_SUB)),
            pltpu.VMEM((3, m_chunk, n_half), jnp.float32),
            pltpu.SemaphoreType.DMA((3, N_SUB)),
            pltpu.SemaphoreType.DMA((3, N_SUB)),
            pltpu.VMEM((N_DEV, 8, 128), jnp.float32),
            pltpu.SemaphoreType.DMA((N_DEV - 1,)),
            pltpu.SemaphoreType.DMA((N_DEV,)),
        ],
        compiler_params=pltpu.CompilerParams(
            collective_id=0,
            vmem_limit_bytes=128 * 1024 * 1024,
        ),
    )(x, w_mat)
